# Initial kernel scaffold; baseline (speedup 1.0000x reference)
import jax
import jax.numpy as jnp
from jax.experimental import pallas as pl


def kernel(xyz, points):
    raise NotImplementedError("write your pallas kernel here")


# speedup vs baseline: 1.2866x; 1.2866x over previous
import functools

import jax
import jax.numpy as jnp
from jax import lax
from jax.experimental import pallas as pl
from jax.experimental.pallas import tpu as pltpu
from jax.experimental.pallas import tpu_sc as plsc

NUM_SAMPLES_ = 32

_NC = 2
_NS = 16
_L = 16

_B, _CX, _N = 16, 3, 4096
_C = 256
_S = NUM_SAMPLES_
_G = _N // _S
_GS = _G * _S

_TILES = _NC * _NS
_CPT = _C // (_TILES // _B)
_RG = 4
_NGROUPS = _CPT // _RG


def _sc_body(points_hbm, xyz_hbm, idx2_hbm, idx1_hbm, idxrep_hbm,
             gpts_hbm, nxyz_hbm, gxyz_hbm,
             idx_v, in_v, out_v, xyz_v, gxyz_v, idx1_v, idxrep_v, nxyz_v):
    cid = lax.axis_index("c")
    sid = lax.axis_index("s")
    wid = sid * _NC + cid
    b = wid // 2
    c0 = (wid % 2) * _CPT

    pltpu.sync_copy(idx2_hbm.at[b], idx_v)

    roff = [jnp.full((_L,), r * _N, dtype=jnp.int32) for r in range(_RG)]

    def group_body(g, carry):
        pltpu.sync_copy(
            points_hbm.at[b, pl.ds((c0 + g * _RG) * _N, _RG * _N)], in_v)

        def inner(j, c2):
            idxv = idx_v[pl.ds(j * _L, _L)]
            for r in range(_RG):
                vals = plsc.load_gather(in_v, [idxv + roff[r]])
                out_v[pl.ds(r * _GS + j * _L, _L)] = vals
            return c2

        lax.fori_loop(0, _GS // _L, inner, 0, unroll=4)
        pltpu.sync_copy(
            out_v, gpts_hbm.at[b, pl.ds((c0 + g * _RG) * _GS, _RG * _GS)])
        return carry

    lax.fori_loop(0, _NGROUPS, group_body, 0)

    @pl.when(wid < _B)
    def _():
        b2 = wid
        pltpu.sync_copy(xyz_hbm.at[b2], xyz_v)
        pltpu.sync_copy(idx1_hbm.at[b2], idx1_v)
        pltpu.sync_copy(idxrep_hbm.at[b2], idxrep_v)
        koff = [jnp.full((_L,), k * _N, dtype=jnp.int32) for k in range(_CX)]
        for j in range(_G // _L):
            idxv = idx1_v[pl.ds(j * _L, _L)]
            for k in range(_CX):
                nxyz_v[pl.ds(k * _G + j * _L, _L)] = plsc.load_gather(
                    xyz_v, [idxv + koff[k]])
        pltpu.sync_copy(nxyz_v, nxyz_hbm.at[b2])

        def gx_inner(j, c2):
            idxv = idxrep_v[pl.ds(j * _L, _L)]
            for k in range(_CX):
                gxyz_v[pl.ds(k * _GS + j * _L, _L)] = plsc.load_gather(
                    xyz_v, [idxv + koff[k]])
            return c2

        lax.fori_loop(0, _GS // _L, gx_inner, 0, unroll=4)
        pltpu.sync_copy(gxyz_v, gxyz_hbm.at[b2])


@jax.jit
def kernel(xyz, points):
    B, C, N = points.shape
    idx_key = jax.random.key(42)
    k1, k2 = jax.random.split(idx_key)
    idx1 = jax.random.randint(k1, (B, _G), 0, N).astype(jnp.int32)
    idx2 = jax.random.randint(k2, (B, _G, _S), 0, N).astype(jnp.int32)
    idx2f = idx2.reshape(B, _GS)
    idxrep = jnp.repeat(idx1, _S, axis=1)

    points_flat = points.reshape(B, C * N)
    xyz_flat = xyz.reshape(B, _CX * N)

    mesh = plsc.VectorSubcoreMesh(
        core_axis_name="c", subcore_axis_name="s",
        num_cores=_NC, num_subcores=_NS)
    run = pl.kernel(
        _sc_body,
        out_type=(
            jax.ShapeDtypeStruct((B, C * _GS), jnp.float32),
            jax.ShapeDtypeStruct((B, _CX * _G), jnp.float32),
            jax.ShapeDtypeStruct((B, _CX * _GS), jnp.float32),
        ),
        mesh=mesh,
        compiler_params=pltpu.CompilerParams(needs_layout_passes=False),
        scratch_types=[
            pltpu.VMEM((_GS,), jnp.int32),
            pltpu.VMEM((_RG * _N,), jnp.float32),
            pltpu.VMEM((_RG * _GS,), jnp.float32),
            pltpu.VMEM((_CX * _N,), jnp.float32),
            pltpu.VMEM((_CX * _GS,), jnp.float32),
            pltpu.VMEM((_G,), jnp.int32),
            pltpu.VMEM((_GS,), jnp.int32),
            pltpu.VMEM((_CX * _G,), jnp.float32),
        ],
    )
    gpts_f, nxyz_f, gxyz_f = run(points_flat, xyz_flat, idx2f, idx1, idxrep)
    new_xyz = nxyz_f.reshape(B, _CX, _G)
    grouped_xyz = gxyz_f.reshape(B, _CX, _G, _S)
    grouped_points = gpts_f.reshape(B, C, _G, _S)
    return (new_xyz, grouped_xyz, grouped_points)


# speedup vs baseline: 1.3959x; 1.0849x over previous
import functools

import jax
import jax.numpy as jnp
from jax import lax
from jax.experimental import pallas as pl
from jax.experimental.pallas import tpu as pltpu
from jax.experimental.pallas import tpu_sc as plsc

NUM_SAMPLES_ = 32

_NC = 2
_NS = 16
_L = 16

_B, _CX, _N = 16, 3, 4096
_C = 256
_S = NUM_SAMPLES_
_G = _N // _S
_GS = _G * _S

_TILES = _NC * _NS
_CPT = _C // (_TILES // _B)
_RG = 4
_NGROUPS = _CPT // _RG


def _sc_body(points_hbm, xyz_hbm, idx2_hbm, idx1_hbm, idxrep_hbm,
             gpts_hbm, nxyz_hbm, gxyz_hbm,
             idx_v, in0, in1, out0, out1, xyz_v, gxyz_v, idx1_v, idxrep_v,
             nxyz_v, sin0, sin1, sout0, sout1):
    cid = lax.axis_index("c")
    sid = lax.axis_index("s")
    wid = sid * _NC + cid
    b = wid // 2
    c0 = (wid % 2) * _CPT

    pltpu.sync_copy(idx2_hbm.at[b], idx_v)

    roff = [jnp.full((_L,), r * _N, dtype=jnp.int32) for r in range(_RG)]
    ins, outs = [in0, in1], [out0, out1]
    sins, souts = [sin0, sin1], [sout0, sout1]

    def src(gg):
        return points_hbm.at[b, pl.ds((c0 + gg * _RG) * _N, _RG * _N)]

    def dst(gg):
        return gpts_hbm.at[b, pl.ds((c0 + gg * _RG) * _GS, _RG * _GS)]

    pltpu.async_copy(src(0), in0, sin0)
    pltpu.async_copy(src(1), in1, sin1)

    def pipe(i, carry):
        for bb in range(2):
            gg = 2 * i + bb
            pltpu.make_async_copy(src(gg), ins[bb], sins[bb]).wait()

            @pl.when(gg >= 2)
            def _():
                pltpu.make_async_copy(outs[bb], dst(gg - 2), souts[bb]).wait()

            def inner(j, c2, bb=bb):
                idxv = idx_v[pl.ds(j * _L, _L)]
                for r in range(_RG):
                    outs[bb][pl.ds(r * _GS + j * _L, _L)] = plsc.load_gather(
                        ins[bb], [idxv + roff[r]])
                return c2

            lax.fori_loop(0, _GS // _L, inner, 0, unroll=4)
            pltpu.async_copy(outs[bb], dst(gg), souts[bb])

            @pl.when(gg + 2 < _NGROUPS)
            def _():
                pltpu.async_copy(src(gg + 2), ins[bb], sins[bb])
        return carry

    lax.fori_loop(0, _NGROUPS // 2, pipe, 0)
    pltpu.make_async_copy(out0, dst(_NGROUPS - 2), sout0).wait()
    pltpu.make_async_copy(out1, dst(_NGROUPS - 1), sout1).wait()

    @pl.when(wid < _B)
    def _():
        b2 = wid
        pltpu.sync_copy(xyz_hbm.at[b2], xyz_v)
        pltpu.sync_copy(idx1_hbm.at[b2], idx1_v)
        pltpu.sync_copy(idxrep_hbm.at[b2], idxrep_v)
        koff = [jnp.full((_L,), k * _N, dtype=jnp.int32) for k in range(_CX)]
        for j in range(_G // _L):
            idxv = idx1_v[pl.ds(j * _L, _L)]
            for k in range(_CX):
                nxyz_v[pl.ds(k * _G + j * _L, _L)] = plsc.load_gather(
                    xyz_v, [idxv + koff[k]])
        pltpu.sync_copy(nxyz_v, nxyz_hbm.at[b2])

        def gx_inner(j, c2):
            idxv = idxrep_v[pl.ds(j * _L, _L)]
            for k in range(_CX):
                gxyz_v[pl.ds(k * _GS + j * _L, _L)] = plsc.load_gather(
                    xyz_v, [idxv + koff[k]])
            return c2

        lax.fori_loop(0, _GS // _L, gx_inner, 0, unroll=4)
        pltpu.sync_copy(gxyz_v, gxyz_hbm.at[b2])


@jax.jit
def kernel(xyz, points):
    B, C, N = points.shape
    idx_key = jax.random.key(42)
    k1, k2 = jax.random.split(idx_key)
    idx1 = jax.random.randint(k1, (B, _G), 0, N).astype(jnp.int32)
    idx2 = jax.random.randint(k2, (B, _G, _S), 0, N).astype(jnp.int32)
    idx2f = idx2.reshape(B, _GS)
    idxrep = jnp.repeat(idx1, _S, axis=1)

    points_flat = points.reshape(B, C * N)
    xyz_flat = xyz.reshape(B, _CX * N)

    mesh = plsc.VectorSubcoreMesh(
        core_axis_name="c", subcore_axis_name="s",
        num_cores=_NC, num_subcores=_NS)
    run = pl.kernel(
        _sc_body,
        out_type=(
            jax.ShapeDtypeStruct((B, C * _GS), jnp.float32),
            jax.ShapeDtypeStruct((B, _CX * _G), jnp.float32),
            jax.ShapeDtypeStruct((B, _CX * _GS), jnp.float32),
        ),
        mesh=mesh,
        compiler_params=pltpu.CompilerParams(needs_layout_passes=False),
        scratch_types=[
            pltpu.VMEM((_GS,), jnp.int32),
            pltpu.VMEM((_RG * _N,), jnp.float32),
            pltpu.VMEM((_RG * _N,), jnp.float32),
            pltpu.VMEM((_RG * _GS,), jnp.float32),
            pltpu.VMEM((_RG * _GS,), jnp.float32),
            pltpu.VMEM((_CX * _N,), jnp.float32),
            pltpu.VMEM((_CX * _GS,), jnp.float32),
            pltpu.VMEM((_G,), jnp.int32),
            pltpu.VMEM((_GS,), jnp.int32),
            pltpu.VMEM((_CX * _G,), jnp.float32),
            pltpu.SemaphoreType.DMA,
            pltpu.SemaphoreType.DMA,
            pltpu.SemaphoreType.DMA,
            pltpu.SemaphoreType.DMA,
        ],
    )
    gpts_f, nxyz_f, gxyz_f = run(points_flat, xyz_flat, idx2f, idx1, idxrep)
    new_xyz = nxyz_f.reshape(B, _CX, _G)
    grouped_xyz = gxyz_f.reshape(B, _CX, _G, _S)
    grouped_points = gpts_f.reshape(B, C, _G, _S)
    return (new_xyz, grouped_xyz, grouped_points)


# speedup vs baseline: 2.1647x; 1.5507x over previous
import functools

import jax
import jax.numpy as jnp
from jax import lax
from jax.experimental import pallas as pl
from jax.experimental.pallas import tpu as pltpu
from jax.experimental.pallas import tpu_sc as plsc

NUM_SAMPLES_ = 32

_NC = 2
_NS = 16
_L = 16

_B, _CX, _N = 16, 3, 4096
_C = 256
_S = NUM_SAMPLES_
_G = _N // _S
_GS = _G * _S

_TILES = _NC * _NS
_CPT = _C // (_TILES // _B)
_RG = 4
_NGROUPS = _CPT // _RG


def _sc_body(points_hbm, xyz_hbm, idx2_hbm, idx1_hbm, idxrep_hbm,
             gpts_hbm, nxyz_hbm, gxyz_hbm,
             idx_v, in0, in1, out0, out1, xyz_v, gxyz_v, idx1_v, idxrep_v,
             nxyz_v, sin0, sin1, sout0, sout1):
    cid = lax.axis_index("c")
    sid = lax.axis_index("s")
    wid = sid * _NC + cid
    b = wid // 2
    c0 = (wid % 2) * _CPT

    pltpu.sync_copy(idx2_hbm.at[b], idx_v)

    roff = [jnp.full((_L,), r * _N, dtype=jnp.int32) for r in range(_RG)]
    ins, outs = [in0, in1], [out0, out1]
    sins, souts = [sin0, sin1], [sout0, sout1]

    def start_in(gg, bb):
        for r in range(_RG):
            pltpu.async_copy(points_hbm.at[b, c0 + gg * _RG + r],
                             ins[bb].at[pl.ds(r * _N, _N)], sins[bb])

    def wait_in(gg, bb):
        for r in range(_RG):
            pltpu.make_async_copy(points_hbm.at[b, c0 + gg * _RG + r],
                                  ins[bb].at[pl.ds(r * _N, _N)],
                                  sins[bb]).wait()

    def start_out(gg, bb):
        for r in range(_RG):
            pltpu.async_copy(outs[bb].at[pl.ds(r * _GS, _GS)],
                             gpts_hbm.at[b, c0 + gg * _RG + r], souts[bb])

    def wait_out(gg, bb):
        for r in range(_RG):
            pltpu.make_async_copy(outs[bb].at[pl.ds(r * _GS, _GS)],
                                  gpts_hbm.at[b, c0 + gg * _RG + r],
                                  souts[bb]).wait()

    start_in(0, 0)
    start_in(1, 1)

    def pipe(i, carry):
        for bb in range(2):
            gg = 2 * i + bb
            wait_in(gg, bb)

            @pl.when(gg >= 2)
            def _():
                wait_out(gg - 2, bb)

            def inner(j, c2, bb=bb):
                idxv = idx_v[pl.ds(j * _L, _L)]
                for r in range(_RG):
                    outs[bb][pl.ds(r * _GS + j * _L, _L)] = plsc.load_gather(
                        ins[bb], [idxv + roff[r]])
                return c2

            lax.fori_loop(0, _GS // _L, inner, 0, unroll=4)
            start_out(gg, bb)

            @pl.when(gg + 2 < _NGROUPS)
            def _():
                start_in(gg + 2, bb)
        return carry

    lax.fori_loop(0, _NGROUPS // 2, pipe, 0)
    wait_out(_NGROUPS - 2, 0)
    wait_out(_NGROUPS - 1, 1)

    @pl.when(wid < _B)
    def _():
        b2 = wid
        for k in range(_CX):
            pltpu.sync_copy(xyz_hbm.at[b2, k], xyz_v.at[pl.ds(k * _N, _N)])
        pltpu.sync_copy(idx1_hbm.at[b2], idx1_v)
        pltpu.sync_copy(idxrep_hbm.at[b2], idxrep_v)
        koff = [jnp.full((_L,), k * _N, dtype=jnp.int32) for k in range(_CX)]
        for j in range(_G // _L):
            idxv = idx1_v[pl.ds(j * _L, _L)]
            for k in range(_CX):
                nxyz_v[pl.ds(k * _G + j * _L, _L)] = plsc.load_gather(
                    xyz_v, [idxv + koff[k]])
        for k in range(_CX):
            pltpu.sync_copy(nxyz_v.at[pl.ds(k * _G, _G)], nxyz_hbm.at[b2, k])

        def gx_inner(j, c2):
            idxv = idxrep_v[pl.ds(j * _L, _L)]
            for k in range(_CX):
                gxyz_v[pl.ds(k * _GS + j * _L, _L)] = plsc.load_gather(
                    xyz_v, [idxv + koff[k]])
            return c2

        lax.fori_loop(0, _GS // _L, gx_inner, 0, unroll=4)
        for k in range(_CX):
            pltpu.sync_copy(gxyz_v.at[pl.ds(k * _GS, _GS)],
                            gxyz_hbm.at[b2, k])


@jax.jit
def kernel(xyz, points):
    B, C, N = points.shape
    idx_key = jax.random.key(42)
    k1, k2 = jax.random.split(idx_key)
    idx1 = jax.random.randint(k1, (B, _G), 0, N).astype(jnp.int32)
    idx2 = jax.random.randint(k2, (B, _G, _S), 0, N).astype(jnp.int32)
    idx2f = idx2.reshape(B, _GS)
    idxrep = jnp.repeat(idx1, _S, axis=1)

    mesh = plsc.VectorSubcoreMesh(
        core_axis_name="c", subcore_axis_name="s",
        num_cores=_NC, num_subcores=_NS)
    run = pl.kernel(
        _sc_body,
        out_type=(
            jax.ShapeDtypeStruct((B, C, _GS), jnp.float32),
            jax.ShapeDtypeStruct((B, _CX, _G), jnp.float32),
            jax.ShapeDtypeStruct((B, _CX, _GS), jnp.float32),
        ),
        mesh=mesh,
        compiler_params=pltpu.CompilerParams(
            needs_layout_passes=False, use_tc_tiling_on_sc=False),
        scratch_types=[
            pltpu.VMEM((_GS,), jnp.int32),
            pltpu.VMEM((_RG * _N,), jnp.float32),
            pltpu.VMEM((_RG * _N,), jnp.float32),
            pltpu.VMEM((_RG * _GS,), jnp.float32),
            pltpu.VMEM((_RG * _GS,), jnp.float32),
            pltpu.VMEM((_CX * _N,), jnp.float32),
            pltpu.VMEM((_CX * _GS,), jnp.float32),
            pltpu.VMEM((_G,), jnp.int32),
            pltpu.VMEM((_GS,), jnp.int32),
            pltpu.VMEM((_CX * _G,), jnp.float32),
            pltpu.SemaphoreType.DMA,
            pltpu.SemaphoreType.DMA,
            pltpu.SemaphoreType.DMA,
            pltpu.SemaphoreType.DMA,
        ],
    )
    gpts, new_xyz, gxyz = run(points, xyz, idx2f, idx1, idxrep)
    grouped_xyz = gxyz.reshape(B, _CX, _G, _S)
    grouped_points = gpts.reshape(B, C, _G, _S)
    return (new_xyz, grouped_xyz, grouped_points)
